# Initial kernel scaffold; baseline (speedup 1.0000x reference)
import jax
import jax.numpy as jnp
from jax.experimental import pallas as pl


def kernel(incidence_matrix, node_embedding, edge_embedding, edge_W, edge_b, edge_ln_g, edge_ln_b, node_W, node_b, node_ln_g, node_ln_b, dec_W, dec_b):
    raise NotImplementedError("write your pallas kernel here")


# speedup vs baseline: 1.0398x; 1.0398x over previous
import jax
import jax.numpy as jnp
from jax.experimental import pallas as pl
from jax.experimental.pallas import tpu as pltpu

_N, _M, _D = 8192, 4096, 64
_BN = 256
_G = _N // _BN


def _ln(h, g, b):
    mu = jnp.mean(h, axis=-1, keepdims=True)
    d = h - mu
    var = jnp.mean(d * d, axis=-1, keepdims=True)
    return d * jax.lax.rsqrt(var + 1e-5) * g + b


def _dtt(a, b):
    return jax.lax.dot_general(a, b, (((0,), (0,)), ((), ())),
                               preferred_element_type=jnp.float32)


def _p1_body(a_ref, n0_ref, e0_ref, wt_ref, b_ref, g_ref, be_ref,
             out_ref, acc_ref):
    i = pl.program_id(0)

    @pl.when(i == 0)
    def _init():
        acc_ref[...] = jnp.zeros_like(acc_ref)

    panel = a_ref[...]
    aug = jnp.concatenate(
        [n0_ref[...], jnp.ones((_BN, 1), jnp.float32)], axis=1)
    acc_ref[...] += _dtt(panel, aug)

    @pl.when(i == _G - 1)
    def _epilogue():
        acc = acc_ref[...]
        edeg = jnp.clip(acc[:, _D:_D + 1], 1e-6, None)
        emsg = acc[:, :_D] / edeg
        e0 = e0_ref[...]
        comb = jnp.concatenate([e0, emsg], axis=1)
        h = jnp.maximum(
            jnp.dot(comb, wt_ref[...], preferred_element_type=jnp.float32)
            + b_ref[...], 0.0)
        e1 = e0 + _ln(h, g_ref[...], be_ref[...])
        out_ref[...] = jnp.concatenate(
            [e1, jnp.ones((_M, 1), jnp.float32)], axis=1)


def _p2_body(a_ref, n0_ref, e1_ref, nwt_ref, nb_ref, ng_ref, nbe_ref,
             ewt_ref, eb_ref, eg_ref, ebe_ref, dw_ref, db_ref,
             out_ref, acc_ref):
    i = pl.program_id(0)

    @pl.when(i == 0)
    def _init():
        acc_ref[...] = jnp.zeros_like(acc_ref)

    panel = a_ref[...]
    tmp = jnp.dot(panel, e1_ref[...], preferred_element_type=jnp.float32)
    ndeg = jnp.clip(tmp[:, _D:_D + 1], 1e-6, None)
    nmsg = tmp[:, :_D] / ndeg
    n0 = n0_ref[...]
    comb = jnp.concatenate([n0, nmsg], axis=1)
    h = jnp.maximum(
        jnp.dot(comb, nwt_ref[...], preferred_element_type=jnp.float32)
        + nb_ref[...], 0.0)
    n1 = n0 + _ln(h, ng_ref[...], nbe_ref[...])
    aug = jnp.concatenate([n1, jnp.ones((_BN, 1), jnp.float32)], axis=1)
    acc_ref[...] += _dtt(panel, aug)

    @pl.when(i == _G - 1)
    def _epilogue():
        acc = acc_ref[...]
        edeg = jnp.clip(acc[:, _D:_D + 1], 1e-6, None)
        emsg = acc[:, :_D] / edeg
        e1 = e1_ref[...][:, :_D]
        comb2 = jnp.concatenate([e1, emsg], axis=1)
        h2 = jnp.maximum(
            jnp.dot(comb2, ewt_ref[...], preferred_element_type=jnp.float32)
            + eb_ref[...], 0.0)
        e2 = e1 + _ln(h2, eg_ref[...], ebe_ref[...])
        logits = jnp.dot(e2, dw_ref[...],
                         preferred_element_type=jnp.float32) + db_ref[...]
        out_ref[...] = jax.nn.sigmoid(0.7 * logits)


def _full(shape):
    return pl.BlockSpec(shape, lambda i: (0, 0))


def kernel(incidence_matrix, node_embedding, edge_embedding, edge_W, edge_b,
           edge_ln_g, edge_ln_b, node_W, node_b, node_ln_g, node_ln_b,
           dec_W, dec_b):
    f32 = jnp.float32
    row2 = lambda x: x.reshape(1, _D).astype(f32)

    e1_aug = pl.pallas_call(
        _p1_body,
        grid=(_G,),
        in_specs=[
            pl.BlockSpec((_BN, _M), lambda i: (i, 0)),
            pl.BlockSpec((_BN, _D), lambda i: (i, 0)),
            _full((_M, _D)),
            _full((2 * _D, _D)),
            _full((1, _D)),
            _full((1, _D)),
            _full((1, _D)),
        ],
        out_specs=_full((_M, _D + 1)),
        out_shape=jax.ShapeDtypeStruct((_M, _D + 1), f32),
        scratch_shapes=[pltpu.VMEM((_M, _D + 1), f32)],
    )(incidence_matrix, node_embedding, edge_embedding,
      edge_W[0].T.astype(f32), row2(edge_b[0]),
      row2(edge_ln_g[0]), row2(edge_ln_b[0]))

    probs = pl.pallas_call(
        _p2_body,
        grid=(_G,),
        in_specs=[
            pl.BlockSpec((_BN, _M), lambda i: (i, 0)),
            pl.BlockSpec((_BN, _D), lambda i: (i, 0)),
            _full((_M, _D + 1)),
            _full((2 * _D, _D)),
            _full((1, _D)),
            _full((1, _D)),
            _full((1, _D)),
            _full((2 * _D, _D)),
            _full((1, _D)),
            _full((1, _D)),
            _full((1, _D)),
            _full((_D, 1)),
            _full((1, 1)),
        ],
        out_specs=_full((_M, 1)),
        out_shape=jax.ShapeDtypeStruct((_M, 1), f32),
        scratch_shapes=[pltpu.VMEM((_M, _D + 1), f32)],
    )(incidence_matrix, node_embedding, e1_aug,
      node_W[0].T.astype(f32), row2(node_b[0]),
      row2(node_ln_g[0]), row2(node_ln_b[0]),
      edge_W[1].T.astype(f32), row2(edge_b[1]),
      row2(edge_ln_g[1]), row2(edge_ln_b[1]),
      dec_W.reshape(_D, 1).astype(f32), dec_b.reshape(1, 1).astype(f32))

    return probs[:, 0]


# speedup vs baseline: 1.2533x; 1.2053x over previous
import jax
import jax.numpy as jnp
from jax.experimental import pallas as pl
from jax.experimental.pallas import tpu as pltpu

_N, _M, _D = 8192, 4096, 64
_BN = 256
_G = _N // _BN


def _ln(h, g, b):
    mu = jnp.mean(h, axis=-1, keepdims=True)
    d = h - mu
    var = jnp.mean(d * d, axis=-1, keepdims=True)
    return d * jax.lax.rsqrt(var + 1e-5) * g + b


def _dtt(a, b):
    return jax.lax.dot_general(a, b, (((0,), (0,)), ((), ())),
                               preferred_element_type=jnp.float32)


def _p1_body(a_ref, n0_ref, e0_ref, wt_ref, b_ref, g_ref, be_ref,
             out_ref, acc_ref):
    i = pl.program_id(0)

    @pl.when(i == 0)
    def _init():
        acc_ref[...] = jnp.zeros_like(acc_ref)

    panel = a_ref[...].astype(jnp.bfloat16)
    aug = jnp.concatenate(
        [n0_ref[...], jnp.ones((_BN, 1), jnp.float32)],
        axis=1).astype(jnp.bfloat16)
    acc_ref[...] += _dtt(panel, aug)

    @pl.when(i == _G - 1)
    def _epilogue():
        acc = acc_ref[...]
        edeg = jnp.clip(acc[:, _D:_D + 1], 1e-6, None)
        emsg = acc[:, :_D] / edeg
        e0 = e0_ref[...]
        comb = jnp.concatenate([e0, emsg], axis=1)
        h = jnp.maximum(
            jnp.dot(comb, wt_ref[...], preferred_element_type=jnp.float32)
            + b_ref[...], 0.0)
        e1 = e0 + _ln(h, g_ref[...], be_ref[...])
        out_ref[...] = jnp.concatenate(
            [e1, jnp.ones((_M, 1), jnp.float32)], axis=1)


def _p2_body(a_ref, n0_ref, e1_ref, nwt_ref, nb_ref, ng_ref, nbe_ref,
             ewt_ref, eb_ref, eg_ref, ebe_ref, dw_ref, db_ref,
             out_ref, acc_ref):
    i = pl.program_id(0)

    @pl.when(i == 0)
    def _init():
        acc_ref[...] = jnp.zeros_like(acc_ref)

    panel = a_ref[...].astype(jnp.bfloat16)
    tmp = jnp.dot(panel, e1_ref[...].astype(jnp.bfloat16),
                  preferred_element_type=jnp.float32)
    ndeg = jnp.clip(tmp[:, _D:_D + 1], 1e-6, None)
    nmsg = tmp[:, :_D] / ndeg
    n0 = n0_ref[...]
    comb = jnp.concatenate([n0, nmsg], axis=1)
    h = jnp.maximum(
        jnp.dot(comb, nwt_ref[...], preferred_element_type=jnp.float32)
        + nb_ref[...], 0.0)
    n1 = n0 + _ln(h, ng_ref[...], nbe_ref[...])
    aug = jnp.concatenate(
        [n1, jnp.ones((_BN, 1), jnp.float32)], axis=1).astype(jnp.bfloat16)
    acc_ref[...] += _dtt(panel, aug)

    @pl.when(i == _G - 1)
    def _epilogue():
        acc = acc_ref[...]
        edeg = jnp.clip(acc[:, _D:_D + 1], 1e-6, None)
        emsg = acc[:, :_D] / edeg
        e1 = e1_ref[...][:, :_D]
        comb2 = jnp.concatenate([e1, emsg], axis=1)
        h2 = jnp.maximum(
            jnp.dot(comb2, ewt_ref[...], preferred_element_type=jnp.float32)
            + eb_ref[...], 0.0)
        e2 = e1 + _ln(h2, eg_ref[...], ebe_ref[...])
        logits = jnp.dot(e2, dw_ref[...],
                         preferred_element_type=jnp.float32) + db_ref[...]
        out_ref[...] = jax.nn.sigmoid(0.7 * logits)


def _full(shape):
    return pl.BlockSpec(shape, lambda i: (0, 0))


def kernel(incidence_matrix, node_embedding, edge_embedding, edge_W, edge_b,
           edge_ln_g, edge_ln_b, node_W, node_b, node_ln_g, node_ln_b,
           dec_W, dec_b):
    f32 = jnp.float32
    row2 = lambda x: x.reshape(1, _D).astype(f32)

    e1_aug = pl.pallas_call(
        _p1_body,
        grid=(_G,),
        in_specs=[
            pl.BlockSpec((_BN, _M), lambda i: (i, 0)),
            pl.BlockSpec((_BN, _D), lambda i: (i, 0)),
            _full((_M, _D)),
            _full((2 * _D, _D)),
            _full((1, _D)),
            _full((1, _D)),
            _full((1, _D)),
        ],
        out_specs=_full((_M, _D + 1)),
        out_shape=jax.ShapeDtypeStruct((_M, _D + 1), f32),
        scratch_shapes=[pltpu.VMEM((_M, _D + 1), f32)],
    )(incidence_matrix, node_embedding, edge_embedding,
      edge_W[0].T.astype(f32), row2(edge_b[0]),
      row2(edge_ln_g[0]), row2(edge_ln_b[0]))

    probs = pl.pallas_call(
        _p2_body,
        grid=(_G,),
        in_specs=[
            pl.BlockSpec((_BN, _M), lambda i: (i, 0)),
            pl.BlockSpec((_BN, _D), lambda i: (i, 0)),
            _full((_M, _D + 1)),
            _full((2 * _D, _D)),
            _full((1, _D)),
            _full((1, _D)),
            _full((1, _D)),
            _full((2 * _D, _D)),
            _full((1, _D)),
            _full((1, _D)),
            _full((1, _D)),
            _full((_D, 1)),
            _full((1, 1)),
        ],
        out_specs=_full((_M, 1)),
        out_shape=jax.ShapeDtypeStruct((_M, 1), f32),
        scratch_shapes=[pltpu.VMEM((_M, _D + 1), f32)],
    )(incidence_matrix, node_embedding, e1_aug,
      node_W[0].T.astype(f32), row2(node_b[0]),
      row2(node_ln_g[0]), row2(node_ln_b[0]),
      edge_W[1].T.astype(f32), row2(edge_b[1]),
      row2(edge_ln_g[1]), row2(edge_ln_b[1]),
      dec_W.reshape(_D, 1).astype(f32), dec_b.reshape(1, 1).astype(f32))

    return probs[:, 0]
